# Initial kernel scaffold; baseline (speedup 1.0000x reference)
import jax
import jax.numpy as jnp
from jax.experimental import pallas as pl


def kernel(points, W_d, b_d, W_a, b_a, W_e, b_e, ln_g, ln_b):
    raise NotImplementedError("write your pallas kernel here")


# speedup vs baseline: 6.6014x; 6.6014x over previous
import numpy as np
import jax
import jax.numpy as jnp
from jax.experimental import pallas as pl
from jax.experimental.pallas import tpu as pltpu

N = 256
HID = 256
NF = HID // 2
INV_SIG_D = 5.0
INV_SIG_A = 1.0 / 15.0
RAD2DEG = float(180.0 / np.pi)
FREQ_SCALE = float(-2.0 * np.log(10000.0) / HID)
BLK = 2048


def _freq_row():
    t = jax.lax.broadcasted_iota(jnp.int32, (1, NF), 1).astype(jnp.float32)
    return jnp.exp(t * FREQ_SCALE)


def _sin_cos_concat(x_col):
    om = x_col * _freq_row()
    return jnp.concatenate([jnp.sin(om), jnp.cos(om)], axis=1)


def _geom_kernel(pts_ref, ptsT_ref, WaT_ref, ba_ref,
                 d_ref, a0_ref, a1_ref, a2_ref, T_ref):
    px = pts_ref[:, 0:1]
    py = pts_ref[:, 1:2]
    pz = pts_ref[:, 2:3]
    rx = ptsT_ref[0:1, :]
    ry = ptsT_ref[1:2, :]
    rz = ptsT_ref[2:3, :]
    ax = rx - px
    ay = ry - py
    az = rz - pz
    bxc = px.astype(jnp.bfloat16).astype(jnp.float32)
    byc = py.astype(jnp.bfloat16).astype(jnp.float32)
    bzc = pz.astype(jnp.bfloat16).astype(jnp.float32)
    bxr = rx.astype(jnp.bfloat16).astype(jnp.float32)
    byr = ry.astype(jnp.bfloat16).astype(jnp.float32)
    bzr = rz.astype(jnp.bfloat16).astype(jnp.float32)
    ab = bxc * bxr + byc * byr + bzc * bzr
    sqc = px * px + py * py + pz * pz
    sqr = rx * rx + ry * ry + rz * rz
    d2 = jnp.maximum((sqc + sqr) - 2.0 * ab, 0.0)
    dist = jnp.sqrt(d2)
    d_ref[...] = dist / jnp.float32(0.2)

    jio = jax.lax.broadcasted_iota(jnp.int32, (N, N), 1)
    iio = jax.lax.broadcasted_iota(jnp.int32, (N, N), 0)
    diag = jio == iio
    BIG = jnp.float32(3.0e38)
    dm = dist
    outs = (None, a0_ref, a1_ref, a2_ref)
    for k in range(4):
        m = jnp.min(dm, axis=1, keepdims=True)
        idxc = jnp.min(jnp.where(dm == m, jio, N), axis=1, keepdims=True)
        oh = jio == idxc
        dm = jnp.where(oh, BIG, dm)
        if k == 0:
            continue
        kx = jnp.sum(jnp.where(oh, rx, 0.0), axis=1, keepdims=True)
        ky = jnp.sum(jnp.where(oh, ry, 0.0), axis=1, keepdims=True)
        kz = jnp.sum(jnp.where(oh, rz, 0.0), axis=1, keepdims=True)
        vx = kx - px
        vy = ky - py
        vz = kz - pz
        cx = vy * az - vz * ay
        cy = vz * ax - vx * az
        cz = vx * ay - vy * ax
        s = jnp.sqrt(cx * cx + cy * cy + cz * cz)
        c = vx * ax + vy * ay + vz * az
        ang = jnp.arctan2(s, c)
        bins = jnp.round(ang * RAD2DEG).astype(jnp.int32)
        allneg = jnp.logical_and(jnp.logical_and(vx < 0, vy < 0), vz < 0)
        dbin = jnp.where(allneg, 180, 0)
        outs[k][...] = jnp.where(diag, dbin, bins)

    r_i = jax.lax.broadcasted_iota(jnp.int32, (N, 1), 0)
    emb = _sin_cos_concat(r_i.astype(jnp.float32) * INV_SIG_A)
    T = jnp.dot(emb, WaT_ref[...], preferred_element_type=jnp.float32)
    T = T + ba_ref[...]
    T_ref[...] = jnp.where(r_i <= 180, T, 0.0)


def _dense_kernel(d_ref, a0_ref, a1_ref, a2_ref, T_ref,
                  WdT_ref, bd_ref, WeT_ref, be_ref, g_ref, b_ref, o_ref):
    demb = jnp.dot(_sin_cos_concat(d_ref[...]), WdT_ref[...],
                   preferred_element_type=jnp.float32) + bd_ref[...]
    rio = jax.lax.broadcasted_iota(jnp.int32, (BLK, HID), 1)
    T = T_ref[...]

    def gat(aref):
        oh = (rio == aref[...]).astype(jnp.float32)
        return jnp.dot(oh, T, preferred_element_type=jnp.float32)

    a_emb = jnp.maximum(jnp.maximum(gat(a0_ref), gat(a1_ref)), gat(a2_ref))
    e = demb + a_emb
    e = jnp.where(e > 0, e, 0.2 * e)
    o = jnp.dot(e, WeT_ref[...], preferred_element_type=jnp.float32) + be_ref[...]
    mu = jnp.mean(o, axis=-1, keepdims=True)
    var = jnp.mean((o - mu) * (o - mu), axis=-1, keepdims=True)
    o_ref[...] = (o - mu) * jax.lax.rsqrt(var + 1e-5) * g_ref[...] + b_ref[...]


def _deinterleave(W):
    Wt = W.T.reshape(NF, 2, HID)
    return jnp.concatenate([Wt[:, 0, :], Wt[:, 1, :]], axis=0)


def kernel(points, W_d, b_d, W_a, b_a, W_e, b_e, ln_g, ln_b):
    B = points.shape[0]
    pts = points[0]
    ptsT = pts.T
    WaT = _deinterleave(W_a)
    WdT = _deinterleave(W_d)
    WeT = W_e.T

    geom = pl.pallas_call(
        _geom_kernel,
        out_shape=(
            jax.ShapeDtypeStruct((N, N), jnp.float32),
            jax.ShapeDtypeStruct((N, N), jnp.int32),
            jax.ShapeDtypeStruct((N, N), jnp.int32),
            jax.ShapeDtypeStruct((N, N), jnp.int32),
            jax.ShapeDtypeStruct((N, HID), jnp.float32),
        ),
    )
    d_idx, a0, a1, a2, T = geom(pts, ptsT, WaT, b_a.reshape(1, HID))

    NN = N * N
    grid = NN // BLK
    col = pl.BlockSpec((BLK, 1), lambda i: (i, 0))
    full = pl.BlockSpec((HID, HID), lambda i: (0, 0))
    row = pl.BlockSpec((1, HID), lambda i: (0, 0))
    dense = pl.pallas_call(
        _dense_kernel,
        grid=(grid,),
        in_specs=[col, col, col, col, full, full, row, full, row, row, row],
        out_specs=pl.BlockSpec((BLK, HID), lambda i: (i, 0)),
        out_shape=jax.ShapeDtypeStruct((NN, HID), jnp.float32),
    )
    out = dense(
        d_idx.reshape(NN, 1), a0.reshape(NN, 1), a1.reshape(NN, 1),
        a2.reshape(NN, 1), T, WdT, b_d.reshape(1, HID), WeT,
        b_e.reshape(1, HID), ln_g.reshape(1, HID), ln_b.reshape(1, HID))
    return out.reshape(B, N, N, HID)


# speedup vs baseline: 8.5768x; 1.2992x over previous
import numpy as np
import jax
import jax.numpy as jnp
from jax.experimental import pallas as pl
from jax.experimental.pallas import tpu as pltpu

N = 256
HID = 256
NF = HID // 2
INV_SIG_D = 5.0
INV_SIG_A = 1.0 / 15.0
RAD2DEG = float(180.0 / np.pi)
FREQ_SCALE = float(-2.0 * np.log(10000.0) / HID)
BLK = 2048


def _freq_row():
    t = jax.lax.broadcasted_iota(jnp.int32, (1, NF), 1).astype(jnp.float32)
    return jnp.exp(t * FREQ_SCALE)


_TWO_OVER_PI = 0.6366197723675814
_PI2_HI = 1.5707855224609375
_PI2_MD = 1.0780334472656250e-05
_PI2_LO = 2.3999487e-08


def _fast_sincos(om):
    mf = jnp.floor(om * _TWO_OVER_PI + 0.5)
    q = mf.astype(jnp.int32)
    r = om - mf * _PI2_HI
    r = r - mf * _PI2_MD
    r = r - mf * _PI2_LO
    r2 = r * r
    sp = r * (1.0 + r2 * (-1.6666654611e-1 + r2 * (8.3321608736e-3
                          + r2 * (-1.9515295891e-4))))
    cp = 1.0 + r2 * (-0.5 + r2 * (4.166664568298827e-2
                     + r2 * (-1.388731625493765e-3
                             + r2 * 2.443315711809948e-5)))
    odd = (q & 1) == 1
    ge2 = (q & 2) == 2
    s_base = jnp.where(odd, cp, sp)
    c_base = jnp.where(odd, sp, cp)
    s = jnp.where(ge2, -s_base, s_base)
    c = jnp.where(odd != ge2, -c_base, c_base)
    return s, c


def _sin_cos_concat(x_col):
    om = x_col * _freq_row()
    s, c = _fast_sincos(om)
    return jnp.concatenate([s, c], axis=1)


def _geom_kernel(pts_ref, ptsT_ref, WaT_ref, ba_ref,
                 d_ref, a0_ref, a1_ref, a2_ref, T_ref):
    px = pts_ref[:, 0:1]
    py = pts_ref[:, 1:2]
    pz = pts_ref[:, 2:3]
    rx = ptsT_ref[0:1, :]
    ry = ptsT_ref[1:2, :]
    rz = ptsT_ref[2:3, :]
    ax = rx - px
    ay = ry - py
    az = rz - pz
    bxc = px.astype(jnp.bfloat16).astype(jnp.float32)
    byc = py.astype(jnp.bfloat16).astype(jnp.float32)
    bzc = pz.astype(jnp.bfloat16).astype(jnp.float32)
    bxr = rx.astype(jnp.bfloat16).astype(jnp.float32)
    byr = ry.astype(jnp.bfloat16).astype(jnp.float32)
    bzr = rz.astype(jnp.bfloat16).astype(jnp.float32)
    ab = bxc * bxr + byc * byr + bzc * bzr
    sqc = px * px + py * py + pz * pz
    sqr = rx * rx + ry * ry + rz * rz
    d2 = jnp.maximum((sqc + sqr) - 2.0 * ab, 0.0)
    dist = jnp.sqrt(d2)
    d_ref[...] = dist / jnp.float32(0.2)

    jio = jax.lax.broadcasted_iota(jnp.int32, (N, N), 1)
    iio = jax.lax.broadcasted_iota(jnp.int32, (N, N), 0)
    diag = jio == iio
    BIG = jnp.float32(3.0e38)
    dm = dist
    outs = (None, a0_ref, a1_ref, a2_ref)
    for k in range(4):
        m = jnp.min(dm, axis=1, keepdims=True)
        idxc = jnp.min(jnp.where(dm == m, jio, N), axis=1, keepdims=True)
        oh = jio == idxc
        dm = jnp.where(oh, BIG, dm)
        if k == 0:
            continue
        kx = jnp.sum(jnp.where(oh, rx, 0.0), axis=1, keepdims=True)
        ky = jnp.sum(jnp.where(oh, ry, 0.0), axis=1, keepdims=True)
        kz = jnp.sum(jnp.where(oh, rz, 0.0), axis=1, keepdims=True)
        vx = kx - px
        vy = ky - py
        vz = kz - pz
        cx = vy * az - vz * ay
        cy = vz * ax - vx * az
        cz = vx * ay - vy * ax
        s = jnp.sqrt(cx * cx + cy * cy + cz * cz)
        c = vx * ax + vy * ay + vz * az
        ang = jnp.arctan2(s, c)
        bins = jnp.round(ang * RAD2DEG).astype(jnp.int32)
        allneg = jnp.logical_and(jnp.logical_and(vx < 0, vy < 0), vz < 0)
        dbin = jnp.where(allneg, 180, 0)
        outs[k][...] = jnp.where(diag, dbin, bins)

    r_i = jax.lax.broadcasted_iota(jnp.int32, (N, 1), 0)
    emb = _sin_cos_concat(r_i.astype(jnp.float32) * INV_SIG_A)
    T = jnp.dot(emb, WaT_ref[...], preferred_element_type=jnp.float32)
    T = T + ba_ref[...]
    T_ref[...] = jnp.where(r_i <= 180, T, 0.0)


def _dense_kernel(d_ref, a0_ref, a1_ref, a2_ref, T_ref,
                  WdT_ref, bd_ref, WeT_ref, be_ref, g_ref, b_ref, o_ref):
    demb = jnp.dot(_sin_cos_concat(d_ref[...]), WdT_ref[...],
                   preferred_element_type=jnp.float32) + bd_ref[...]
    rio = jax.lax.broadcasted_iota(jnp.int32, (BLK, HID), 1)
    T = T_ref[...]

    def gat(aref):
        oh = (rio == aref[...]).astype(jnp.float32)
        return jnp.dot(oh, T, preferred_element_type=jnp.float32)

    a_emb = jnp.maximum(jnp.maximum(gat(a0_ref), gat(a1_ref)), gat(a2_ref))
    e = demb + a_emb
    e = jnp.where(e > 0, e, 0.2 * e)
    o = jnp.dot(e, WeT_ref[...], preferred_element_type=jnp.float32) + be_ref[...]
    mu = jnp.mean(o, axis=-1, keepdims=True)
    var = jnp.mean((o - mu) * (o - mu), axis=-1, keepdims=True)
    o_ref[...] = (o - mu) * jax.lax.rsqrt(var + 1e-5) * g_ref[...] + b_ref[...]


def _deinterleave(W):
    Wt = W.T.reshape(NF, 2, HID)
    return jnp.concatenate([Wt[:, 0, :], Wt[:, 1, :]], axis=0)


def kernel(points, W_d, b_d, W_a, b_a, W_e, b_e, ln_g, ln_b):
    B = points.shape[0]
    pts = points[0]
    ptsT = pts.T
    WaT = _deinterleave(W_a)
    WdT = _deinterleave(W_d)
    WeT = W_e.T

    geom = pl.pallas_call(
        _geom_kernel,
        out_shape=(
            jax.ShapeDtypeStruct((N, N), jnp.float32),
            jax.ShapeDtypeStruct((N, N), jnp.int32),
            jax.ShapeDtypeStruct((N, N), jnp.int32),
            jax.ShapeDtypeStruct((N, N), jnp.int32),
            jax.ShapeDtypeStruct((N, HID), jnp.float32),
        ),
    )
    d_idx, a0, a1, a2, T = geom(pts, ptsT, WaT, b_a.reshape(1, HID))

    NN = N * N
    grid = NN // BLK
    col = pl.BlockSpec((BLK, 1), lambda i: (i, 0))
    full = pl.BlockSpec((HID, HID), lambda i: (0, 0))
    row = pl.BlockSpec((1, HID), lambda i: (0, 0))
    dense = pl.pallas_call(
        _dense_kernel,
        grid=(grid,),
        in_specs=[col, col, col, col, full, full, row, full, row, row, row],
        out_specs=pl.BlockSpec((BLK, HID), lambda i: (i, 0)),
        out_shape=jax.ShapeDtypeStruct((NN, HID), jnp.float32),
    )
    out = dense(
        d_idx.reshape(NN, 1), a0.reshape(NN, 1), a1.reshape(NN, 1),
        a2.reshape(NN, 1), T, WdT, b_d.reshape(1, HID), WeT,
        b_e.reshape(1, HID), ln_g.reshape(1, HID), ln_b.reshape(1, HID))
    return out.reshape(B, N, N, HID)
